# baseline (speedup 1.0000x reference)
import jax
import jax.numpy as jnp
from jax import lax
from jax.experimental import pallas as pl
from jax.experimental.pallas import tpu as pltpu
from jax.experimental.pallas import tpu_sc as plsc

VOCAB = 1000000
EMBED = 64
BATCH = 4096
SEQ = 200

NC = 2
NS = 16
NW = NC * NS

FBLK = 128
NFULL = VOCAB // FBLK
FPW = NFULL // NW
NEXTRA = NFULL - FPW * NW

GCH = 128
NSCH = SEQ
NBG = 4
NBO = 2


def _worker_id():
    return lax.axis_index("s") * NC + lax.axis_index("c")


def _fmt_body(tin, tail, fout, vbuf, obuf, tail_v, gsem, psem):
    wid = _worker_id()
    iota = jax.lax.iota(jnp.int32, 16)
    zeros16 = iota * 0
    fvecs = [iota + 16 * i for i in range(4)]

    def fire_g(c, slot, ntok):
        pltpu.async_copy(
            tin.at[:, pl.ds(c * FBLK, ntok)],
            vbuf.at[slot, :, pl.ds(0, ntok)],
            gsem.at[slot],
        )

    def wait_g(slot, ntok):
        pltpu.make_async_copy(
            tin.at[:, pl.ds(0, ntok)],
            vbuf.at[slot, :, pl.ds(0, ntok)],
            gsem.at[slot],
        ).wait()

    def fire_p(c, slot, ntok):
        pltpu.async_copy(
            obuf.at[slot, pl.ds(0, ntok // 2)],
            fout.at[pl.ds(c * (FBLK // 2), ntok // 2)],
            psem.at[slot],
        )

    def wait_p(slot, ntok):
        pltpu.make_async_copy(
            obuf.at[slot, pl.ds(0, ntok // 2)],
            fout.at[pl.ds(0, ntok // 2)],
            psem.at[slot],
        ).wait()

    def transpose(slot, ntok):
        src = vbuf.at[slot]
        dst = obuf.at[slot]

        @pl.loop(0, ntok, unroll=4)
        def _(tt):
            tvec = zeros16 + tt
            r2 = tt >> 1
            col = (tt & 1) * 64
            for i in range(4):
                val = plsc.load_gather(src, [fvecs[i], tvec])
                dst[r2, pl.ds(col + 16 * i, 16)] = val

    c0 = wid * FPW
    fire_g(c0, 0, FBLK)
    fire_g(c0 + 1, 1, FBLK)
    for b in (0, 1):
        wait_g(b, FBLK)
        transpose(b, FBLK)
        fire_p(c0 + b, b, FBLK)
        fire_g(c0 + b + 2, b, FBLK)

    @pl.loop(c0 + 2, c0 + FPW - 2, step=2)
    def _(ci):
        for b in (0, 1):
            c = ci + b
            wait_g(b, FBLK)
            wait_p(b, FBLK)
            transpose(b, FBLK)
            fire_p(c, b, FBLK)
            fire_g(c + 2, b, FBLK)

    for b in (0, 1):
        c = c0 + FPW - 2 + b
        wait_g(b, FBLK)
        wait_p(b, FBLK)
        transpose(b, FBLK)
        fire_p(c, b, FBLK)
    for b in (0, 1):
        wait_p(b, FBLK)

    @pl.when(wid < NEXTRA)
    def _():
        c = FPW * NW + wid
        fire_g(c, 0, FBLK)
        wait_g(0, FBLK)
        transpose(0, FBLK)
        fire_p(c, 0, FBLK)
        wait_p(0, FBLK)

    @pl.when(wid == NEXTRA)
    def _():
        pltpu.sync_copy(tail, tail_v)
        pltpu.sync_copy(tail_v, fout.at[pl.ds(NFULL * (FBLK // 2), 32)])


def _gat_body(table_hbm, idx_hbm, out_hbm, idx_v, rows_v, obuf, gsem, psem):
    wid = _worker_id()
    iota = jax.lax.iota(jnp.int32, 16)
    zeros16 = iota * 0
    gvecs = [iota + 16 * g for g in range(8)]

    pltpu.sync_copy(idx_hbm.at[:, wid], idx_v)

    def fire_g(s, slot):
        pltpu.async_copy(
            table_hbm.at[idx_v.at[s >> 3, s & 7]],
            rows_v.at[slot],
            gsem.at[slot],
        )

    def wait_g(slot):
        pltpu.make_async_copy(
            table_hbm.at[idx_v.at[0, 0]], rows_v.at[slot], gsem.at[slot]
        ).wait()

    def fire_p(s, slot):
        pltpu.async_copy(obuf.at[slot], out_hbm.at[s, :, wid], psem.at[slot])

    def wait_p(slot):
        pltpu.make_async_copy(
            obuf.at[slot], out_hbm.at[0, :, wid], psem.at[slot]
        ).wait()

    def transpose(slotg, sloto):
        src = rows_v.at[slotg]
        dst = obuf.at[sloto]

        @pl.loop(0, EMBED, unroll=4)
        def _(e):
            evec = zeros16 + e
            e8 = e >> 3
            e1 = e & 7
            for g in range(8):
                val = plsc.load_gather(src, [gvecs[g], evec])
                dst[e8, e1, pl.ds(16 * g, 16)] = val

    for slot in range(NBG):
        fire_g(slot, slot)

    for s in (0, 1):
        wait_g(s)
        transpose(s, s)
        fire_p(s, s)
        fire_g(s + NBG, s)

    @pl.loop(2, NSCH - 6, step=NBG)
    def _(si):
        for b in range(NBG):
            s = si + b
            slot = (2 + b) % NBG
            sloto = b % NBO
            wait_g(slot)
            wait_p(sloto)
            transpose(slot, sloto)
            fire_p(s, sloto)
            fire_g(s + NBG, slot)

    for s in range(NSCH - 6, NSCH):
        slot = s % NBG
        sloto = s % NBO
        wait_g(slot)
        wait_p(sloto)
        transpose(slot, sloto)
        fire_p(s, sloto)
        if s + NBG < NSCH:
            fire_g(s + NBG, slot)
    for s in (NSCH - 2, NSCH - 1):
        wait_p(s % NBO)


@jax.jit
def _emb2(table, tokens):
    mesh = plsc.VectorSubcoreMesh(core_axis_name="c", subcore_axis_name="s")
    fmt = pl.kernel(
        _fmt_body,
        out_type=jax.ShapeDtypeStruct((VOCAB // 2, 128), jnp.float32),
        mesh=mesh,
        scratch_types=[
            pltpu.VMEM((2, EMBED, 128), jnp.float32),
            pltpu.VMEM((2, EMBED, 128), jnp.float32),
            pltpu.VMEM((32, 128), jnp.float32),
            pltpu.SemaphoreType.DMA((2,)),
            pltpu.SemaphoreType.DMA((2,)),
        ],
        compiler_params=pltpu.CompilerParams(use_tc_tiling_on_sc=True, needs_layout_passes=False),
    )
    tail = table[NFULL * FBLK:].reshape(32, 128)
    tflat = fmt(table.T, tail).reshape(VOCAB, EMBED)

    idx4 = tokens.reshape(32, 128, 25, 8).transpose(2, 0, 3, 1)

    gat = pl.kernel(
        _gat_body,
        out_type=jax.ShapeDtypeStruct((SEQ, 8, 32, 8, 128), jnp.float32),
        mesh=mesh,
        scratch_types=[
            pltpu.VMEM((25, 8, 128), jnp.int32),
            pltpu.VMEM((NBG, GCH, EMBED), jnp.float32),
            pltpu.VMEM((NBO, 8, 8, 128), jnp.float32),
            pltpu.SemaphoreType.DMA((NBG,)),
            pltpu.SemaphoreType.DMA((NBO,)),
        ],
        compiler_params=pltpu.CompilerParams(use_tc_tiling_on_sc=False, needs_layout_passes=False),
    )
    out5 = gat(tflat, idx4)
    return out5.transpose(2, 4, 0, 1, 3).reshape(BATCH, SEQ, EMBED)


def kernel(input_tokens, table):
    return _emb2(table, input_tokens.astype(jnp.int32))


# speedup vs baseline: 1.8850x; 1.8850x over previous
import jax
import jax.numpy as jnp
from jax import lax
from jax.experimental import pallas as pl
from jax.experimental.pallas import tpu as pltpu
from jax.experimental.pallas import tpu_sc as plsc

VOCAB = 1000000
EMBED = 64
BATCH = 4096
SEQ = 200

NC = 2
NS = 16
NW = NC * NS

FBLK = 128
NFULL = VOCAB // FBLK
FPW = NFULL // NW
NEXTRA = NFULL - FPW * NW

GCH = 128
NSCH = SEQ
NBG = 4
NBO = 2


def _worker_id():
    return lax.axis_index("s") * NC + lax.axis_index("c")


def _fmt_body(tin, tail, fout, vbuf, obuf, tail_v, gsem, psem):
    wid = _worker_id()
    iota = jax.lax.iota(jnp.int32, 16)
    zeros16 = iota * 0
    fvecs = [iota + 16 * i for i in range(4)]

    def fire_g(c, slot, ntok):
        pltpu.async_copy(
            tin.at[:, pl.ds(c * FBLK, ntok)],
            vbuf.at[slot, :, pl.ds(0, ntok)],
            gsem.at[slot],
        )

    def wait_g(slot, ntok):
        pltpu.make_async_copy(
            tin.at[:, pl.ds(0, ntok)],
            vbuf.at[slot, :, pl.ds(0, ntok)],
            gsem.at[slot],
        ).wait()

    def fire_p(c, slot, ntok):
        pltpu.async_copy(
            obuf.at[slot, pl.ds(0, ntok // 2)],
            fout.at[pl.ds(c * (FBLK // 2), ntok // 2)],
            psem.at[slot],
        )

    def wait_p(slot, ntok):
        pltpu.make_async_copy(
            obuf.at[slot, pl.ds(0, ntok // 2)],
            fout.at[pl.ds(0, ntok // 2)],
            psem.at[slot],
        ).wait()

    def transpose(slot, ntok):
        src = vbuf.at[slot]
        dst = obuf.at[slot]

        @plsc.parallel_loop(0, ntok, unroll=4)
        def _(tt):
            tvec = zeros16 + tt
            r2 = tt >> 1
            col = (tt & 1) * 64
            for i in range(4):
                val = plsc.load_gather(src, [fvecs[i], tvec])
                dst[r2, pl.ds(col + 16 * i, 16)] = val

    c0 = wid * FPW
    fire_g(c0, 0, FBLK)
    fire_g(c0 + 1, 1, FBLK)
    for b in (0, 1):
        wait_g(b, FBLK)
        transpose(b, FBLK)
        fire_p(c0 + b, b, FBLK)
        fire_g(c0 + b + 2, b, FBLK)

    @pl.loop(c0 + 2, c0 + FPW - 2, step=2)
    def _(ci):
        for b in (0, 1):
            c = ci + b
            wait_g(b, FBLK)
            wait_p(b, FBLK)
            transpose(b, FBLK)
            fire_p(c, b, FBLK)
            fire_g(c + 2, b, FBLK)

    for b in (0, 1):
        c = c0 + FPW - 2 + b
        wait_g(b, FBLK)
        wait_p(b, FBLK)
        transpose(b, FBLK)
        fire_p(c, b, FBLK)
    for b in (0, 1):
        wait_p(b, FBLK)

    @pl.when(wid < NEXTRA)
    def _():
        c = FPW * NW + wid
        fire_g(c, 0, FBLK)
        wait_g(0, FBLK)
        transpose(0, FBLK)
        fire_p(c, 0, FBLK)
        wait_p(0, FBLK)

    @pl.when(wid == NEXTRA)
    def _():
        pltpu.sync_copy(tail, tail_v)
        pltpu.sync_copy(tail_v, fout.at[pl.ds(NFULL * (FBLK // 2), 32)])


def _gat_body(table_hbm, idx_hbm, out_hbm, idx_v, rows_v, obuf, gsem, psem):
    wid = _worker_id()
    iota = jax.lax.iota(jnp.int32, 16)
    zeros16 = iota * 0
    gvecs = [iota + 16 * g for g in range(8)]

    pltpu.sync_copy(idx_hbm.at[:, wid], idx_v)

    def fire_g(s, slot):
        pltpu.async_copy(
            table_hbm.at[idx_v.at[s >> 3, s & 7]],
            rows_v.at[slot],
            gsem.at[slot],
        )

    def wait_g(slot):
        pltpu.make_async_copy(
            table_hbm.at[idx_v.at[0, 0]], rows_v.at[slot], gsem.at[slot]
        ).wait()

    def fire_p(s, slot):
        pltpu.async_copy(obuf.at[slot], out_hbm.at[s, :, wid], psem.at[slot])

    def wait_p(slot):
        pltpu.make_async_copy(
            obuf.at[slot], out_hbm.at[0, :, wid], psem.at[slot]
        ).wait()

    def transpose(slotg, sloto):
        src = rows_v.at[slotg]
        dst = obuf.at[sloto]

        @plsc.parallel_loop(0, EMBED, unroll=4)
        def _(e):
            evec = zeros16 + e
            e8 = e >> 3
            e1 = e & 7
            for g in range(8):
                val = plsc.load_gather(src, [gvecs[g], evec])
                dst[e8, e1, pl.ds(16 * g, 16)] = val

    for slot in range(NBG):
        fire_g(slot, slot)

    for s in (0, 1):
        wait_g(s)
        transpose(s, s)
        fire_p(s, s)
        fire_g(s + NBG, s)

    @pl.loop(2, NSCH - 6, step=NBG)
    def _(si):
        for b in range(NBG):
            s = si + b
            slot = (2 + b) % NBG
            sloto = b % NBO
            wait_g(slot)
            wait_p(sloto)
            transpose(slot, sloto)
            fire_p(s, sloto)
            fire_g(s + NBG, slot)

    for s in range(NSCH - 6, NSCH):
        slot = s % NBG
        sloto = s % NBO
        wait_g(slot)
        wait_p(sloto)
        transpose(slot, sloto)
        fire_p(s, sloto)
        if s + NBG < NSCH:
            fire_g(s + NBG, slot)
    for s in (NSCH - 2, NSCH - 1):
        wait_p(s % NBO)


@jax.jit
def _emb2(table, tokens):
    mesh = plsc.VectorSubcoreMesh(core_axis_name="c", subcore_axis_name="s")
    fmt = pl.kernel(
        _fmt_body,
        out_type=jax.ShapeDtypeStruct((VOCAB // 2, 128), jnp.float32),
        mesh=mesh,
        scratch_types=[
            pltpu.VMEM((2, EMBED, 128), jnp.float32),
            pltpu.VMEM((2, EMBED, 128), jnp.float32),
            pltpu.VMEM((32, 128), jnp.float32),
            pltpu.SemaphoreType.DMA((2,)),
            pltpu.SemaphoreType.DMA((2,)),
        ],
        compiler_params=pltpu.CompilerParams(use_tc_tiling_on_sc=True, needs_layout_passes=False, disable_bounds_checks=True),
    )
    tail = table[NFULL * FBLK:].reshape(32, 128)
    tflat = fmt(table.T, tail).reshape(VOCAB, EMBED)

    idx4 = tokens.reshape(32, 128, 25, 8).transpose(2, 0, 3, 1)

    gat = pl.kernel(
        _gat_body,
        out_type=jax.ShapeDtypeStruct((SEQ, 8, 32, 8, 128), jnp.float32),
        mesh=mesh,
        scratch_types=[
            pltpu.VMEM((25, 8, 128), jnp.int32),
            pltpu.VMEM((NBG, GCH, EMBED), jnp.float32),
            pltpu.VMEM((NBO, 8, 8, 128), jnp.float32),
            pltpu.SemaphoreType.DMA((NBG,)),
            pltpu.SemaphoreType.DMA((NBO,)),
        ],
        compiler_params=pltpu.CompilerParams(use_tc_tiling_on_sc=False, needs_layout_passes=False, disable_bounds_checks=True),
    )
    out5 = gat(tflat, idx4)
    return out5.transpose(2, 4, 0, 1, 3).reshape(BATCH, SEQ, EMBED)


def kernel(input_tokens, table):
    return _emb2(table, input_tokens.astype(jnp.int32))


# speedup vs baseline: 2.8860x; 1.5310x over previous
import jax
import jax.numpy as jnp
from jax import lax
from jax.experimental import pallas as pl
from jax.experimental.pallas import tpu as pltpu
from jax.experimental.pallas import tpu_sc as plsc

VOCAB = 1000000
EMBED = 64
BATCH = 4096
SEQ = 200

NC = 2
NS = 16
NW = NC * NS

FBLK = 128
NFULL = VOCAB // FBLK
FPW = NFULL // NW
NEXTRA = NFULL - FPW * NW

GCH = 128
NSCH = SEQ
NBG = 4
NBO = 2


def _worker_id():
    return lax.axis_index("s") * NC + lax.axis_index("c")


def _fmt_body(tin, tail, fout, vbuf, obuf, tail_v, gsem, psem):
    wid = _worker_id()
    iota = jax.lax.iota(jnp.int32, 16)
    zeros16 = iota * 0
    fvecs = [iota + 16 * i for i in range(4)]

    def fire_g(c, slot, ntok):
        pltpu.async_copy(
            tin.at[:, pl.ds(c * FBLK, ntok)],
            vbuf.at[slot, :, pl.ds(0, ntok)],
            gsem.at[slot],
        )

    def wait_g(slot, ntok):
        pltpu.make_async_copy(
            tin.at[:, pl.ds(0, ntok)],
            vbuf.at[slot, :, pl.ds(0, ntok)],
            gsem.at[slot],
        ).wait()

    def fire_p(c, slot, ntok):
        pltpu.async_copy(
            obuf.at[slot, pl.ds(0, ntok // 2)],
            fout.at[pl.ds(c * (FBLK // 2), ntok // 2)],
            psem.at[slot],
        )

    def wait_p(slot, ntok):
        pltpu.make_async_copy(
            obuf.at[slot, pl.ds(0, ntok // 2)],
            fout.at[pl.ds(0, ntok // 2)],
            psem.at[slot],
        ).wait()

    def transpose(slot, ntok):
        src = vbuf.at[slot]
        dst = obuf.at[slot]

        @plsc.parallel_loop(0, ntok, unroll=4)
        def _(tt):
            tvec = zeros16 + tt
            r2 = tt >> 1
            col = (tt & 1) * 64
            for i in range(4):
                val = plsc.load_gather(src, [fvecs[i], tvec])
                dst[r2, pl.ds(col + 16 * i, 16)] = val

    c0 = wid * FPW
    fire_g(c0, 0, FBLK)
    fire_g(c0 + 1, 1, FBLK)
    for b in (0, 1):
        wait_g(b, FBLK)
        transpose(b, FBLK)
        fire_p(c0 + b, b, FBLK)
        fire_g(c0 + b + 2, b, FBLK)

    @pl.loop(c0 + 2, c0 + FPW - 2, step=2)
    def _(ci):
        for b in (0, 1):
            c = ci + b
            wait_g(b, FBLK)
            wait_p(b, FBLK)
            transpose(b, FBLK)
            fire_p(c, b, FBLK)
            fire_g(c + 2, b, FBLK)

    for b in (0, 1):
        c = c0 + FPW - 2 + b
        wait_g(b, FBLK)
        wait_p(b, FBLK)
        transpose(b, FBLK)
        fire_p(c, b, FBLK)
    for b in (0, 1):
        wait_p(b, FBLK)

    @pl.when(wid < NEXTRA)
    def _():
        c = FPW * NW + wid
        fire_g(c, 0, FBLK)
        wait_g(0, FBLK)
        transpose(0, FBLK)
        fire_p(c, 0, FBLK)
        wait_p(0, FBLK)

    @pl.when(wid == NEXTRA)
    def _():
        pltpu.sync_copy(tail, tail_v)
        pltpu.sync_copy(tail_v, fout.at[pl.ds(NFULL * (FBLK // 2), 32)])


def _gat_body(table_hbm, idx_hbm, out_hbm, idx_v, rows_v, obuf, gsem, psem):
    wid = _worker_id()
    iota = jax.lax.iota(jnp.int32, 16)
    zeros16 = iota * 0
    gvecs = [iota + 16 * g for g in range(8)]

    pltpu.sync_copy(idx_hbm.at[:, wid], idx_v)

    def fire_g(s, slot):
        pltpu.async_copy(
            table_hbm.at[idx_v.at[s >> 3, s & 7]],
            rows_v.at[slot],
            gsem.at[slot],
        )

    def wait_g(slot):
        pltpu.make_async_copy(
            table_hbm.at[idx_v.at[0, 0]], rows_v.at[slot], gsem.at[slot]
        ).wait()

    def fire_p(s, slot):
        pltpu.async_copy(
            obuf.at[slot, :, :, pl.ds(0, 128)],
            out_hbm.at[s, :, wid],
            psem.at[slot],
        )

    def wait_p(slot):
        pltpu.make_async_copy(
            obuf.at[slot, :, :, pl.ds(0, 128)],
            out_hbm.at[0, :, wid],
            psem.at[slot],
        ).wait()

    e8vs = [(iota + 16 * i) >> 3 for i in range(4)]
    e1vs = [(iota + 16 * i) & 7 for i in range(4)]

    def transpose(slotg, sloto):
        src = rows_v.at[slotg]
        dst = obuf.at[sloto]

        @plsc.parallel_loop(0, GCH, unroll=4)
        def _(tt):
            ttv = zeros16 + tt
            for i in range(4):
                val = src[tt, pl.ds(16 * i, 16)]
                plsc.store_scatter(dst, [e8vs[i], e1vs[i], ttv], val)

    for slot in range(NBG):
        fire_g(slot, slot)

    for s in (0, 1):
        wait_g(s)
        transpose(s, s)
        fire_p(s, s)
        fire_g(s + NBG, s)

    @pl.loop(2, NSCH - 6, step=NBG)
    def _(si):
        for b in range(NBG):
            s = si + b
            slot = (2 + b) % NBG
            sloto = b % NBO
            wait_g(slot)
            wait_p(sloto)
            transpose(slot, sloto)
            fire_p(s, sloto)
            fire_g(s + NBG, slot)

    for s in range(NSCH - 6, NSCH):
        slot = s % NBG
        sloto = s % NBO
        wait_g(slot)
        wait_p(sloto)
        transpose(slot, sloto)
        fire_p(s, sloto)
        if s + NBG < NSCH:
            fire_g(s + NBG, slot)
    for s in (NSCH - 2, NSCH - 1):
        wait_p(s % NBO)


@jax.jit
def _emb2(table, tokens):
    mesh = plsc.VectorSubcoreMesh(core_axis_name="c", subcore_axis_name="s")
    fmt = pl.kernel(
        _fmt_body,
        out_type=jax.ShapeDtypeStruct((VOCAB // 2, 128), jnp.float32),
        mesh=mesh,
        scratch_types=[
            pltpu.VMEM((2, EMBED, 129), jnp.float32),
            pltpu.VMEM((2, EMBED, 128), jnp.float32),
            pltpu.VMEM((32, 128), jnp.float32),
            pltpu.SemaphoreType.DMA((2,)),
            pltpu.SemaphoreType.DMA((2,)),
        ],
        compiler_params=pltpu.CompilerParams(use_tc_tiling_on_sc=True, needs_layout_passes=False, disable_bounds_checks=True),
    )
    tail = table[NFULL * FBLK:].reshape(32, 128)
    tflat = fmt(table.T, tail).reshape(VOCAB, EMBED)

    idx4 = tokens.reshape(32, 128, 25, 8).transpose(2, 0, 3, 1)

    gat = pl.kernel(
        _gat_body,
        out_type=jax.ShapeDtypeStruct((SEQ, 8, 32, 8, 128), jnp.float32),
        mesh=mesh,
        scratch_types=[
            pltpu.VMEM((25, 8, 128), jnp.int32),
            pltpu.VMEM((NBG, GCH, EMBED), jnp.float32),
            pltpu.VMEM((NBO, 8, 8, 129), jnp.float32),
            pltpu.SemaphoreType.DMA((NBG,)),
            pltpu.SemaphoreType.DMA((NBO,)),
        ],
        compiler_params=pltpu.CompilerParams(use_tc_tiling_on_sc=False, needs_layout_passes=False, disable_bounds_checks=True),
    )
    out5 = gat(tflat, idx4)
    return out5.transpose(2, 4, 0, 1, 3).reshape(BATCH, SEQ, EMBED)


def kernel(input_tokens, table):
    return _emb2(table, input_tokens.astype(jnp.int32))
